# baseline (speedup 1.0000x reference)
import jax
import jax.numpy as jnp
from jax.experimental import pallas as pl
from jax.experimental.pallas import tpu as pltpu

_VOCAB = 2048
_SBLK = 512
_NSLOT = 8


def _body(ids_ref, out_ref, scratch, sems):
    i = pl.program_id(0)
    nblk = pl.num_programs(0)
    slot = jax.lax.rem(i, _NSLOT)

    @pl.when(i >= _NSLOT)
    def _():
        pltpu.make_async_copy(scratch.at[slot], out_ref.at[i - _NSLOT],
                              sems.at[slot]).wait()

    ids = ids_ref[0]
    pred = ids % 3 + 1
    iota = jax.lax.broadcasted_iota(jnp.int32, (_SBLK, _VOCAB), 1)
    scratch[slot] = jnp.where(iota == pred, 50.0, -50.0)
    pltpu.make_async_copy(scratch.at[slot], out_ref.at[i],
                          sems.at[slot]).start()

    @pl.when(i == nblk - 1)
    def _():
        for j in range(_NSLOT):
            tail = nblk - _NSLOT + j
            pltpu.make_async_copy(scratch.at[jax.lax.rem(tail, _NSLOT)],
                                  out_ref.at[tail],
                                  sems.at[jax.lax.rem(tail, _NSLOT)]).wait()


def kernel(input_ids):
    b, s = input_ids.shape
    n = b * s
    nblk = n // _SBLK
    ids3 = input_ids.reshape(nblk, _SBLK, 1)
    out = pl.pallas_call(
        _body,
        grid=(nblk,),
        in_specs=[pl.BlockSpec((1, _SBLK, 1), lambda i: (i, 0, 0))],
        out_specs=pl.BlockSpec(memory_space=pl.ANY),
        out_shape=jax.ShapeDtypeStruct((nblk, _SBLK, _VOCAB), jnp.float32),
        scratch_shapes=[
            pltpu.VMEM((_NSLOT, _SBLK, _VOCAB), jnp.float32),
            pltpu.SemaphoreType.DMA((_NSLOT,)),
        ],
    )(ids3)
    return out.reshape(b, s, _VOCAB)
